# Initial kernel scaffold; baseline (speedup 1.0000x reference)
import jax
import jax.numpy as jnp
from jax.experimental import pallas as pl


def kernel(unknown, known, unknow_feats, known_feats, W1, W2):
    raise NotImplementedError("write your pallas kernel here")


# speedup vs baseline: 26.4658x; 26.4658x over previous
import functools
import jax
import jax.numpy as jnp
from jax.experimental import pallas as pl
from jax.experimental.pallas import tpu as pltpu

_B, _N, _M, _C1, _C2 = 8, 4096, 1024, 128, 256
_NBLK = 512
_BIG = 3.0e38


def _fp_body(u_ref, kt_ref, uft_ref, kft_ref, w1_ref, w2_ref, out_ref):
    u = u_ref[0]
    kt = kt_ref[0]

    d2 = jnp.zeros((_NBLK, _M), jnp.float32)
    for d in range(3):
        diff = u[:, d:d + 1] - kt[d:d + 1, :]
        d2 = d2 + diff * diff

    ids = jax.lax.broadcasted_iota(jnp.int32, (_NBLK, _M), 1)
    cur = d2
    mins = []
    onehots = []
    for _ in range(3):
        m = jnp.min(cur, axis=1, keepdims=True)
        i = jnp.min(jnp.where(cur == m, ids, _M), axis=1, keepdims=True)
        oh = ids == i
        mins.append(m)
        onehots.append(oh)
        cur = jnp.where(oh, _BIG, cur)

    r1 = 1.0 / (mins[0] + 1e-8)
    r2 = 1.0 / (mins[1] + 1e-8)
    r3 = 1.0 / (mins[2] + 1e-8)
    norm = r1 + r2 + r3
    w1 = r1 / norm
    w2 = r2 / norm
    w3 = r3 / norm

    zero = jnp.zeros((_NBLK, _M), jnp.float32)
    s = jnp.where(onehots[0], w1, zero)
    s = jnp.where(onehots[1], w2, s)
    s = jnp.where(onehots[2], w3, s)

    interp = jnp.dot(s, kft_ref[0], preferred_element_type=jnp.float32)
    x = jnp.concatenate([interp, uft_ref[0]], axis=1)
    h = jnp.maximum(jnp.dot(x, w1_ref[...],
                            preferred_element_type=jnp.float32), 0.0)
    h = jnp.maximum(jnp.dot(h, w2_ref[...],
                            preferred_element_type=jnp.float32), 0.0)
    out_ref[0] = h.T


@jax.jit
def kernel(unknown, known, unknow_feats, known_feats, W1, W2):
    kt = known.swapaxes(1, 2)
    uft = unknow_feats.swapaxes(1, 2)
    kft = known_feats.swapaxes(1, 2)

    grid = (_B, _N // _NBLK)
    return pl.pallas_call(
        _fp_body,
        grid=grid,
        in_specs=[
            pl.BlockSpec((1, _NBLK, 3), lambda b, n: (b, n, 0)),
            pl.BlockSpec((1, 3, _M), lambda b, n: (b, 0, 0)),
            pl.BlockSpec((1, _NBLK, _C1), lambda b, n: (b, n, 0)),
            pl.BlockSpec((1, _M, _C2), lambda b, n: (b, 0, 0)),
            pl.BlockSpec((_C1 + _C2, 256), lambda b, n: (0, 0)),
            pl.BlockSpec((256, 256), lambda b, n: (0, 0)),
        ],
        out_specs=pl.BlockSpec((1, 256, _NBLK), lambda b, n: (b, 0, n)),
        out_shape=jax.ShapeDtypeStruct((_B, 256, _N), jnp.float32),
    )(unknown, kt, uft, kft, W1, W2)
